# Initial kernel scaffold; baseline (speedup 1.0000x reference)
import jax
import jax.numpy as jnp
from jax.experimental import pallas as pl


def kernel(x, a, W1, as1, ad1, b1, W2, as2, ad2, b2, W3, as3, ad3, b3, gW, gb, aW1, ab1, ag1, abt1, aW2, ab2, ag2, abt2, aW3, ab3, ag3, abt3, fW1, fb1, fg1, fbt1, fW2, fb2, fg2, fbt2, fW3, fb3, edge_index, batch_index):
    raise NotImplementedError("write your pallas kernel here")


# speedup vs baseline: 1.0224x; 1.0224x over previous
import functools

import jax
import jax.numpy as jnp
from jax.experimental import pallas as pl
from jax.experimental.pallas import tpu as pltpu


def _mm_body(x_ref, w_ref, o_ref):
    o_ref[...] = jnp.dot(x_ref[...], w_ref[...],
                         preferred_element_type=jnp.float32)


def _mm(x, w, bm=1000):
    M, K = x.shape
    _, N = w.shape
    return pl.pallas_call(
        _mm_body,
        grid=(M // bm,),
        in_specs=[pl.BlockSpec((bm, K), lambda i: (i, 0)),
                  pl.BlockSpec((K, N), lambda i: (0, 0))],
        out_specs=pl.BlockSpec((bm, N), lambda i: (i, 0)),
        out_shape=jax.ShapeDtypeStruct((M, N), jnp.float32),
    )(x, w)


def _bn(x, g, b, eps=1e-5):
    m = jnp.mean(x, axis=0)
    v = jnp.var(x, axis=0)
    return (x - m) / jnp.sqrt(v + eps) * g + b


def _gat(x, ei, W, asrc, adst, bias, H, C):
    N = x.shape[0]
    h = _mm(x, W).reshape(N, H, C)
    es = jnp.sum(h * asrc[None], axis=-1)
    ed = jnp.sum(h * adst[None], axis=-1)
    src, dst = ei[0], ei[1]
    e = jax.nn.leaky_relu(es[src] + ed[dst], 0.2)
    emax = jax.ops.segment_max(e, dst, num_segments=N)
    emax = jax.lax.stop_gradient(jnp.where(jnp.isfinite(emax), emax, 0.0))
    ex = jnp.exp(e - emax[dst])
    den = jax.ops.segment_sum(ex, dst, num_segments=N)
    alpha = ex / (den[dst] + 1e-16)
    out = jax.ops.segment_sum(h[src] * alpha[..., None], dst, num_segments=N)
    return out.reshape(N, H * C) + bias


def kernel(x, a, W1, as1, ad1, b1, W2, as2, ad2, b2, W3, as3, ad3, b3, gW, gb, aW1, ab1, ag1, abt1, aW2, ab2, ag2, abt2, aW3, ab3, ag3, abt3, fW1, fb1, fg1, fbt1, fW2, fb2, fg2, fbt2, fW3, fb3, edge_index, batch_index):
    N = x.shape[0]
    B = a.shape[0]
    loops = jnp.arange(N, dtype=edge_index.dtype)
    ei = jnp.concatenate([edge_index, jnp.stack([loops, loops])], axis=1)
    h = jax.nn.relu(_gat(x, ei, W1, as1, ad1, b1, 8, 16))
    h = jax.nn.relu(_gat(h, ei, W2, as2, ad2, b2, 8, 16))
    h = jax.nn.relu(_gat(h, ei, W3, as3, ad3, b3, 1, 128))
    gate = (h @ gW + gb)[:, 0]
    gmax = jax.ops.segment_max(gate, batch_index, num_segments=B)
    gmax = jax.lax.stop_gradient(jnp.where(jnp.isfinite(gmax), gmax, 0.0))
    gexp = jnp.exp(gate - gmax[batch_index])
    gden = jax.ops.segment_sum(gexp, batch_index, num_segments=B)
    w = gexp / (gden[batch_index] + 1e-16)
    V = jax.ops.segment_sum(h * w[:, None], batch_index, num_segments=B)
    am = _bn(jax.nn.relu(a @ aW1 + ab1), ag1, abt1)
    am = _bn(jax.nn.relu(am @ aW2 + ab2), ag2, abt2)
    am = _bn(jax.nn.relu(am @ aW3 + ab3), ag3, abt3)
    z = jnp.concatenate([V, am], axis=1)
    z = _bn(jax.nn.relu(z @ fW1 + fb1), fg1, fbt1)
    z = _bn(jax.nn.relu(z @ fW2 + fb2), fg2, fbt2)
    return (z @ fW3 + fb3)[:, 0]
